# baseline (speedup 1.0000x reference)
import dataclasses
import functools

import jax
import jax.numpy as jnp
from jax import lax
from jax.experimental import pallas as pl
from jax.experimental.pallas import tpu as pltpu
from jax.experimental.pallas import tpu_sc as plsc

B = 16384
FEAT_DIM = 256
H = 128
H2 = H // 2
L = 50
LH = L // 2

NC = 2
NS = 16
NW = NC * NS
B_PER_W = B // NW
CH = 16
N_CHUNKS = B_PER_W // CH
IDC = 128
N_IDC = B_PER_W // IDC

_MESH = plsc.VectorSubcoreMesh(
    core_axis_name="c", subcore_axis_name="s", num_cores=NC, num_subcores=NS
)


def _sc_text_body(text_hbm, text_table_hbm, psum_hbm,
                  tidx_v, rows_v, acc_v, sems):
    (sem_g0, sem_g1, sem_x0, sem_x1, sem_o0, sem_o1) = sems
    sem_g = (sem_g0, sem_g1)
    sem_x = (sem_x0, sem_x1)
    sem_o = (sem_o0, sem_o1)

    wid = lax.axis_index("s") * NC + lax.axis_index("c")
    base = wid * B_PER_W

    for b in range(2):
        pltpu.sync_copy(text_hbm.at[pl.ds((base + b * CH) * L, CH * L)],
                        tidx_v[b])
        pltpu.async_copy(text_table_hbm.at[tidx_v[b]], rows_v[b], sem_g[b])

    @pl.loop(0, N_CHUNKS, step=2)
    def _(c):
        for b in range(2):
            c2 = c + b
            pltpu.make_async_copy(
                text_table_hbm.at[tidx_v[b]], rows_v[b], sem_g[b]).wait()

            @pl.when(c2 + 2 < N_CHUNKS)
            def _():
                pltpu.async_copy(
                    text_hbm.at[pl.ds((base + (c2 + 2) * CH) * L, CH * L)],
                    tidx_v[b], sem_x[b])

            @pl.when(c2 >= 2)
            def _():
                pltpu.make_async_copy(
                    acc_v[b], psum_hbm.at[pl.ds(base, CH)], sem_o[b]).wait()

            @pl.loop(0, CH)
            def _(e):
                def add_row(l, accs):
                    r = e * L + l
                    lo = tuple(
                        accs[h] + plsc.bitcast(
                            rows_v[b][r, pl.ds(h * 16, 16)], jnp.bfloat16)
                        for h in range(4)
                    )
                    hi = tuple(
                        accs[4 + h] + plsc.bitcast(
                            rows_v[b][r + LH, pl.ds(h * 16, 16)],
                            jnp.bfloat16)
                        for h in range(4)
                    )
                    return lo + hi

                accs = lax.fori_loop(
                    0, LH, add_row,
                    tuple(jnp.zeros((32,), jnp.bfloat16) for _ in range(8)),
                )
                for h in range(8):
                    acc_v[b][e, pl.ds(h * 16, 16)] = plsc.bitcast(
                        accs[h], jnp.int32)

            pltpu.async_copy(
                acc_v[b], psum_hbm.at[pl.ds(base + c2 * CH, CH)], sem_o[b])

            @pl.when(c2 + 2 < N_CHUNKS)
            def _():
                pltpu.make_async_copy(
                    text_hbm.at[pl.ds((base + (c2 + 2) * CH) * L, CH * L)],
                    tidx_v[b], sem_x[b]).wait()
                pltpu.async_copy(
                    text_table_hbm.at[tidx_v[b]], rows_v[b], sem_g[b])

    for b in range(2):
        pltpu.make_async_copy(
            acc_v[b], psum_hbm.at[pl.ds(base, CH)], sem_o[b]).wait()


def _sc_text(text_flat, text_table_i32):
    out_type = jax.ShapeDtypeStruct((B, H), jnp.int32)
    scratch = [
        (pltpu.VMEM((CH * L,), jnp.int32),) * 2,
        (pltpu.VMEM((CH * L, H2), jnp.int32),) * 2,
        (pltpu.VMEM((CH, H), jnp.int32),) * 2,
        (pltpu.SemaphoreType.DMA,) * 6,
    ]
    cp = pltpu.CompilerParams()
    if "needs_layout_passes" in pltpu.CompilerParams.__dataclass_fields__:
        cp = dataclasses.replace(cp, needs_layout_passes=False)
    if "use_tc_tiling_on_sc" in pltpu.CompilerParams.__dataclass_fields__:
        cp = dataclasses.replace(cp, use_tc_tiling_on_sc=False)
    return pl.kernel(
        _sc_text_body, out_type=out_type, mesh=_MESH, scratch_types=scratch,
        compiler_params=cp,
    )(text_flat, text_table_i32)


def _sc_id_body(ids_hbm, id_table_hbm, pid_hbm, iidx_v, idrows_v, sem):
    wid = lax.axis_index("s") * NC + lax.axis_index("c")
    base = wid * B_PER_W

    @pl.loop(0, N_IDC)
    def _(k):
        row0 = base + k * IDC
        pltpu.sync_copy(ids_hbm.at[pl.ds(row0, IDC)], iidx_v)
        pltpu.async_copy(id_table_hbm.at[iidx_v], idrows_v, sem).wait()
        pltpu.sync_copy(idrows_v, pid_hbm.at[pl.ds(row0, IDC)])


def _sc_id(ids, id_table):
    out_type = jax.ShapeDtypeStruct((B, H), jnp.float32)
    scratch = [
        pltpu.VMEM((IDC,), jnp.int32),
        pltpu.VMEM((IDC, H), jnp.float32),
        pltpu.SemaphoreType.DMA,
    ]
    return pl.kernel(
        _sc_id_body, out_type=out_type, mesh=_MESH, scratch_types=scratch,
    )(ids, id_table)


BLK = 1024


def _mm_body(feat_ref, w_ref, b_ref, out_ref):
    out_ref[...] = lax.dot_general(
        feat_ref[...], w_ref[...], (((1,), (1,)), ((), ())),
        preferred_element_type=jnp.float32,
    ) + b_ref[...]


def _mm(feat, W, b2):
    return pl.pallas_call(
        _mm_body,
        grid=(B // BLK,),
        in_specs=[
            pl.BlockSpec((BLK, FEAT_DIM), lambda i: (i, 0)),
            pl.BlockSpec((H, FEAT_DIM), lambda i: (0, 0)),
            pl.BlockSpec((1, H), lambda i: (0, 0)),
        ],
        out_specs=pl.BlockSpec((BLK, H), lambda i: (i, 0)),
        out_shape=jax.ShapeDtypeStruct((B, H), jnp.float32),
    )(feat, W, b2)


def _combine_body(mm_ref, pid_ref, pa_ref, pb_ref, len_ref, out_ref):
    recip = 1.0 / len_ref[...]
    psum = pa_ref[...].astype(jnp.float32) + pb_ref[...].astype(jnp.float32)
    out_ref[...] = mm_ref[...] + pid_ref[...] + psum * recip


def _combine(mm, pid, pa, pb, len2):
    return pl.pallas_call(
        _combine_body,
        grid=(B // BLK,),
        in_specs=[
            pl.BlockSpec((BLK, H), lambda i: (i, 0)),
            pl.BlockSpec((BLK, H), lambda i: (i, 0)),
            pl.BlockSpec((BLK, H), lambda i: (i, 0)),
            pl.BlockSpec((BLK, H), lambda i: (i, 0)),
            pl.BlockSpec((BLK, 1), lambda i: (i, 0)),
        ],
        out_specs=pl.BlockSpec((BLK, H), lambda i: (i, 0)),
        out_shape=jax.ShapeDtypeStruct((B, H), jnp.float32),
    )(mm, pid, pa, pb, len2)


@jax.jit
def _run(feat, ids, text_flat, len2, W, b2, id_table, text_table_i32):
    psum_i32 = _sc_text(text_flat, text_table_i32)
    pid = _sc_id(ids, id_table)
    mm = _mm(feat, W, b2)
    pbf = lax.bitcast_convert_type(psum_i32, jnp.bfloat16)
    pa = pbf[:, :H2].reshape(B, H)
    pb = pbf[:, H2:].reshape(B, H)
    return _combine(mm, pid, pa, pb, len2)


def kernel(feat, ids, text, text_len, W, b, id_table, text_table):
    ids = ids.astype(jnp.int32)
    text_flat = text.astype(jnp.int32).reshape(B * L)
    len2 = text_len.astype(jnp.float32).reshape(B, 1)
    b2 = b.reshape(1, H)
    tt16 = lax.bitcast_convert_type(
        text_table.astype(jnp.bfloat16), jnp.uint16)
    lo = tt16[:, 0::2].astype(jnp.uint32)
    hi = tt16[:, 1::2].astype(jnp.uint32)
    text_table_i32 = lax.bitcast_convert_type(
        lo | (hi << 16), jnp.int32)
    return _run(feat, ids, text_flat, len2, W, b2, id_table,
                text_table_i32)


# speedup vs baseline: 4.1774x; 4.1774x over previous
import dataclasses
import functools

import jax
import jax.numpy as jnp
from jax import lax
from jax.experimental import pallas as pl
from jax.experimental.pallas import tpu as pltpu
from jax.experimental.pallas import tpu_sc as plsc

B = 16384
FEAT_DIM = 256
H = 128
H2 = H // 2
L = 50
LH = L // 2

NC = 2
NS = 16
NW = NC * NS
B_PER_W = B // NW
CH = 16
N_CHUNKS = B_PER_W // CH
IDC = 128
N_IDC = B_PER_W // IDC

_MESH = plsc.VectorSubcoreMesh(
    core_axis_name="c", subcore_axis_name="s", num_cores=NC, num_subcores=NS
)


def _sc_text_body(text_hbm, text_table_hbm, psum_hbm,
                  tidx_v, rows_v, acc_v, sems):
    (sem_g0, sem_g1, sem_x0, sem_x1, sem_o0, sem_o1) = sems
    sem_g = (sem_g0, sem_g1)
    sem_x = (sem_x0, sem_x1)
    sem_o = (sem_o0, sem_o1)

    wid = lax.axis_index("s") * NC + lax.axis_index("c")
    base = wid * B_PER_W

    for b in range(2):
        pltpu.sync_copy(text_hbm.at[pl.ds((base + b * CH) * L, CH * L)],
                        tidx_v[b])
        pltpu.async_copy(text_table_hbm.at[tidx_v[b]], rows_v[b], sem_g[b])

    @pl.loop(0, N_CHUNKS, step=2)
    def _(c):
        for b in range(2):
            c2 = c + b
            pltpu.make_async_copy(
                text_table_hbm.at[tidx_v[b]], rows_v[b], sem_g[b]).wait()

            @pl.when(c2 + 2 < N_CHUNKS)
            def _():
                pltpu.async_copy(
                    text_hbm.at[pl.ds((base + (c2 + 2) * CH) * L, CH * L)],
                    tidx_v[b], sem_x[b])

            @pl.when(c2 >= 2)
            def _():
                pltpu.make_async_copy(
                    acc_v[b], psum_hbm.at[pl.ds(base, CH)], sem_o[b]).wait()

            @pl.loop(0, CH)
            def _(e):
                def add_row(l, accs):
                    r = e * L + l
                    lo = tuple(
                        accs[h] + plsc.bitcast(
                            rows_v[b][r, pl.ds(h * 16, 16)], jnp.bfloat16)
                        for h in range(4)
                    )
                    hi = tuple(
                        accs[4 + h] + plsc.bitcast(
                            rows_v[b][r + LH, pl.ds(h * 16, 16)],
                            jnp.bfloat16)
                        for h in range(4)
                    )
                    return lo + hi

                accs = lax.fori_loop(
                    0, LH, add_row,
                    tuple(jnp.zeros((32,), jnp.bfloat16) for _ in range(8)),
                )
                for h in range(8):
                    acc_v[b][e, pl.ds(h * 16, 16)] = plsc.bitcast(
                        accs[h], jnp.int32)

            pltpu.async_copy(
                acc_v[b], psum_hbm.at[pl.ds(base + c2 * CH, CH)], sem_o[b])

            @pl.when(c2 + 2 < N_CHUNKS)
            def _():
                pltpu.make_async_copy(
                    text_hbm.at[pl.ds((base + (c2 + 2) * CH) * L, CH * L)],
                    tidx_v[b], sem_x[b]).wait()
                pltpu.async_copy(
                    text_table_hbm.at[tidx_v[b]], rows_v[b], sem_g[b])

    for b in range(2):
        pltpu.make_async_copy(
            acc_v[b], psum_hbm.at[pl.ds(base, CH)], sem_o[b]).wait()


def _sc_text(text_flat, text_table_i32):
    out_type = jax.ShapeDtypeStruct((B, H), jnp.int32)
    scratch = [
        (pltpu.VMEM((CH * L,), jnp.int32),) * 2,
        (pltpu.VMEM((CH * L, H2), jnp.int32),) * 2,
        (pltpu.VMEM((CH, H), jnp.int32),) * 2,
        (pltpu.SemaphoreType.DMA,) * 6,
    ]
    cp = pltpu.CompilerParams()
    if "needs_layout_passes" in pltpu.CompilerParams.__dataclass_fields__:
        cp = dataclasses.replace(cp, needs_layout_passes=False)
    if "use_tc_tiling_on_sc" in pltpu.CompilerParams.__dataclass_fields__:
        cp = dataclasses.replace(cp, use_tc_tiling_on_sc=False)
    return pl.kernel(
        _sc_text_body, out_type=out_type, mesh=_MESH, scratch_types=scratch,
        compiler_params=cp,
    )(text_flat, text_table_i32)


def _sc_id_body(ids_hbm, id_table_hbm, pid_hbm, iidx_v, idrows_v, sem):
    wid = lax.axis_index("s") * NC + lax.axis_index("c")
    base = wid * B_PER_W

    @pl.loop(0, N_IDC)
    def _(k):
        row0 = base + k * IDC
        pltpu.sync_copy(ids_hbm.at[pl.ds(row0, IDC)], iidx_v)
        pltpu.async_copy(id_table_hbm.at[iidx_v], idrows_v, sem).wait()
        pltpu.sync_copy(idrows_v, pid_hbm.at[pl.ds(row0, IDC)])


def _sc_id(ids, id_table):
    out_type = jax.ShapeDtypeStruct((B, H), jnp.float32)
    scratch = [
        pltpu.VMEM((IDC,), jnp.int32),
        pltpu.VMEM((IDC, H), jnp.float32),
        pltpu.SemaphoreType.DMA,
    ]
    return pl.kernel(
        _sc_id_body, out_type=out_type, mesh=_MESH, scratch_types=scratch,
    )(ids, id_table)


BLK = 1024


def _mm_body(feat_ref, w_ref, b_ref, out_ref):
    out_ref[...] = lax.dot_general(
        feat_ref[...], w_ref[...], (((1,), (1,)), ((), ())),
        preferred_element_type=jnp.float32,
    ) + b_ref[...]


def _mm(feat, W, b2):
    return pl.pallas_call(
        _mm_body,
        grid=(B // BLK,),
        in_specs=[
            pl.BlockSpec((BLK, FEAT_DIM), lambda i: (i, 0)),
            pl.BlockSpec((H, FEAT_DIM), lambda i: (0, 0)),
            pl.BlockSpec((1, H), lambda i: (0, 0)),
        ],
        out_specs=pl.BlockSpec((BLK, H), lambda i: (i, 0)),
        out_shape=jax.ShapeDtypeStruct((B, H), jnp.float32),
    )(feat, W, b2)


def _unpack_f32(w):
    lo = lax.bitcast_convert_type(w << 16, jnp.float32)
    hi = lax.bitcast_convert_type(
        w & jnp.int32(-65536), jnp.float32)
    return lo, hi


def _combine_body(mm_ref, pid_ref, ps_ref, len_ref, out_ref):
    recip = 1.0 / len_ref[...]
    wa = ps_ref[:, :H2]
    wb = ps_ref[:, H2:]
    a_lo, a_hi = _unpack_f32(wa)
    b_lo, b_hi = _unpack_f32(wb)
    psum = jnp.concatenate([a_lo + b_lo, a_hi + b_hi], axis=1)
    out_ref[...] = mm_ref[...] + pid_ref[...] + psum * recip


def _combine(mm, pid, psum_i32, len2):
    return pl.pallas_call(
        _combine_body,
        grid=(B // BLK,),
        in_specs=[
            pl.BlockSpec((BLK, H), lambda i: (i, 0)),
            pl.BlockSpec((BLK, H), lambda i: (i, 0)),
            pl.BlockSpec((BLK, H), lambda i: (i, 0)),
            pl.BlockSpec((BLK, 1), lambda i: (i, 0)),
        ],
        out_specs=pl.BlockSpec((BLK, H), lambda i: (i, 0)),
        out_shape=jax.ShapeDtypeStruct((B, H), jnp.float32),
    )(mm, pid, psum_i32, len2)


@jax.jit
def _run(feat, ids, text_flat, len2, W, b2, id_table, text_table_i32):
    psum_i32 = _sc_text(text_flat, text_table_i32)
    pid = _sc_id(ids, id_table)
    mm = _mm(feat, W, b2)
    return _combine(mm, pid, psum_i32, len2)


def kernel(feat, ids, text, text_len, W, b, id_table, text_table):
    ids = ids.astype(jnp.int32)
    text_flat = text.astype(jnp.int32).reshape(B * L)
    len2 = text_len.astype(jnp.float32).reshape(B, 1)
    b2 = b.reshape(1, H)
    tt16 = lax.bitcast_convert_type(
        text_table.astype(jnp.bfloat16), jnp.uint16)
    lo = tt16[:, :H2].astype(jnp.uint32)
    hi = tt16[:, H2:].astype(jnp.uint32)
    text_table_i32 = lax.bitcast_convert_type(lo | (hi << 16), jnp.int32)
    return _run(feat, ids, text_flat, len2, W, b2, id_table,
                text_table_i32)


# speedup vs baseline: 4.1840x; 1.0016x over previous
import dataclasses
import functools

import jax
import jax.numpy as jnp
from jax import lax
from jax.experimental import pallas as pl
from jax.experimental.pallas import tpu as pltpu
from jax.experimental.pallas import tpu_sc as plsc

B = 16384
FEAT_DIM = 256
H = 128
H2 = H // 2
L = 50
LH = L // 2

NC = 2
NS = 16
NW = NC * NS
B_PER_W = B // NW
CH = 16
N_CHUNKS = B_PER_W // CH
IDC = 128
N_IDC = B_PER_W // IDC

_MESH = plsc.VectorSubcoreMesh(
    core_axis_name="c", subcore_axis_name="s", num_cores=NC, num_subcores=NS
)


def _sc_text_body(text_hbm, text_table_hbm, psum_hbm,
                  tidx_v, rows_v, acc_v, sems):
    (sem_g0, sem_g1, sem_x0, sem_x1, sem_o0, sem_o1) = sems
    sem_g = (sem_g0, sem_g1)
    sem_x = (sem_x0, sem_x1)
    sem_o = (sem_o0, sem_o1)

    wid = lax.axis_index("s") * NC + lax.axis_index("c")
    base = wid * B_PER_W

    for b in range(2):
        pltpu.sync_copy(text_hbm.at[pl.ds((base + b * CH) * L, CH * L)],
                        tidx_v[b])
        pltpu.async_copy(text_table_hbm.at[tidx_v[b]], rows_v[b], sem_g[b])

    @pl.loop(0, N_CHUNKS, step=2)
    def _(c):
        for b in range(2):
            c2 = c + b
            pltpu.make_async_copy(
                text_table_hbm.at[tidx_v[b]], rows_v[b], sem_g[b]).wait()

            @pl.when(c2 + 2 < N_CHUNKS)
            def _():
                pltpu.async_copy(
                    text_hbm.at[pl.ds((base + (c2 + 2) * CH) * L, CH * L)],
                    tidx_v[b], sem_x[b])

            @pl.when(c2 >= 2)
            def _():
                pltpu.make_async_copy(
                    acc_v[b], psum_hbm.at[pl.ds(base, CH)], sem_o[b]).wait()

            @pl.loop(0, CH)
            def _(e):
                def add_row(l, accs):
                    r = e * L + l
                    lo = tuple(
                        accs[h] + plsc.bitcast(
                            rows_v[b][r, pl.ds(h * 16, 16)], jnp.bfloat16)
                        for h in range(4)
                    )
                    hi = tuple(
                        accs[4 + h] + plsc.bitcast(
                            rows_v[b][r + LH, pl.ds(h * 16, 16)],
                            jnp.bfloat16)
                        for h in range(4)
                    )
                    return lo + hi

                accs = lax.fori_loop(
                    0, LH, add_row,
                    tuple(jnp.zeros((32,), jnp.bfloat16) for _ in range(8)),
                )
                for h in range(8):
                    acc_v[b][e, pl.ds(h * 16, 16)] = plsc.bitcast(
                        accs[h], jnp.int32)

            pltpu.async_copy(
                acc_v[b], psum_hbm.at[pl.ds(base + c2 * CH, CH)], sem_o[b])

            @pl.when(c2 + 2 < N_CHUNKS)
            def _():
                pltpu.make_async_copy(
                    text_hbm.at[pl.ds((base + (c2 + 2) * CH) * L, CH * L)],
                    tidx_v[b], sem_x[b]).wait()
                pltpu.async_copy(
                    text_table_hbm.at[tidx_v[b]], rows_v[b], sem_g[b])

    for b in range(2):
        pltpu.make_async_copy(
            acc_v[b], psum_hbm.at[pl.ds(base, CH)], sem_o[b]).wait()


def _sc_text(text_flat, text_table_i32):
    out_type = jax.ShapeDtypeStruct((B, H), jnp.int32)
    scratch = [
        (pltpu.VMEM((CH * L,), jnp.int32),) * 2,
        (pltpu.VMEM((CH * L, H2), jnp.int32),) * 2,
        (pltpu.VMEM((CH, H), jnp.int32),) * 2,
        (pltpu.SemaphoreType.DMA,) * 6,
    ]
    cp = pltpu.CompilerParams()
    if "needs_layout_passes" in pltpu.CompilerParams.__dataclass_fields__:
        cp = dataclasses.replace(cp, needs_layout_passes=False)
    if "use_tc_tiling_on_sc" in pltpu.CompilerParams.__dataclass_fields__:
        cp = dataclasses.replace(cp, use_tc_tiling_on_sc=False)
    return pl.kernel(
        _sc_text_body, out_type=out_type, mesh=_MESH, scratch_types=scratch,
        compiler_params=cp,
    )(text_flat, text_table_i32)


def _sc_id_body(ids_hbm, id_table_hbm, pid_hbm, iidx_v, idrows_v, sem):
    wid = lax.axis_index("s") * NC + lax.axis_index("c")
    base = wid * B_PER_W

    @pl.loop(0, N_IDC)
    def _(k):
        row0 = base + k * IDC
        pltpu.sync_copy(ids_hbm.at[pl.ds(row0, IDC)], iidx_v)
        pltpu.async_copy(id_table_hbm.at[iidx_v], idrows_v, sem).wait()
        pltpu.sync_copy(idrows_v, pid_hbm.at[pl.ds(row0, IDC)])


def _sc_id(ids, id_table):
    out_type = jax.ShapeDtypeStruct((B, H), jnp.float32)
    scratch = [
        pltpu.VMEM((IDC,), jnp.int32),
        pltpu.VMEM((IDC, H), jnp.float32),
        pltpu.SemaphoreType.DMA,
    ]
    return pl.kernel(
        _sc_id_body, out_type=out_type, mesh=_MESH, scratch_types=scratch,
    )(ids, id_table)


BLK = 1024


def _mm_body(feat_ref, w_ref, b_ref, out_ref):
    out_ref[...] = lax.dot_general(
        feat_ref[...], w_ref[...], (((1,), (1,)), ((), ())),
        preferred_element_type=jnp.float32,
    ) + b_ref[...]


def _mm(feat, W, b2):
    return pl.pallas_call(
        _mm_body,
        grid=(B // BLK,),
        compiler_params=pltpu.CompilerParams(
            dimension_semantics=("parallel",)),
        in_specs=[
            pl.BlockSpec((BLK, FEAT_DIM), lambda i: (i, 0)),
            pl.BlockSpec((H, FEAT_DIM), lambda i: (0, 0)),
            pl.BlockSpec((1, H), lambda i: (0, 0)),
        ],
        out_specs=pl.BlockSpec((BLK, H), lambda i: (i, 0)),
        out_shape=jax.ShapeDtypeStruct((B, H), jnp.float32),
    )(feat, W, b2)


def _unpack_f32(w):
    lo = lax.bitcast_convert_type(w << 16, jnp.float32)
    hi = lax.bitcast_convert_type(
        w & jnp.int32(-65536), jnp.float32)
    return lo, hi


def _combine_body(mm_ref, pid_ref, ps_ref, len_ref, out_ref):
    recip = 1.0 / len_ref[...]
    wa = ps_ref[:, :H2]
    wb = ps_ref[:, H2:]
    a_lo, a_hi = _unpack_f32(wa)
    b_lo, b_hi = _unpack_f32(wb)
    psum = jnp.concatenate([a_lo + b_lo, a_hi + b_hi], axis=1)
    out_ref[...] = mm_ref[...] + pid_ref[...] + psum * recip


def _combine(mm, pid, psum_i32, len2):
    return pl.pallas_call(
        _combine_body,
        grid=(B // BLK,),
        compiler_params=pltpu.CompilerParams(
            dimension_semantics=("parallel",)),
        in_specs=[
            pl.BlockSpec((BLK, H), lambda i: (i, 0)),
            pl.BlockSpec((BLK, H), lambda i: (i, 0)),
            pl.BlockSpec((BLK, H), lambda i: (i, 0)),
            pl.BlockSpec((BLK, 1), lambda i: (i, 0)),
        ],
        out_specs=pl.BlockSpec((BLK, H), lambda i: (i, 0)),
        out_shape=jax.ShapeDtypeStruct((B, H), jnp.float32),
    )(mm, pid, psum_i32, len2)


@jax.jit
def _run(feat, ids, text_flat, text_len, W, b, id_table, text_table_i32):
    psum_i32 = _sc_text(text_flat, text_table_i32)
    pid = _sc_id(ids, id_table)
    len2 = text_len.astype(jnp.float32).reshape(B, 1)
    b2 = b.reshape(1, H)
    mm = _mm(feat, W, b2)
    return _combine(mm, pid, psum_i32, len2)


def kernel(feat, ids, text, text_len, W, b, id_table, text_table):
    ids = ids.astype(jnp.int32)
    text_flat = text.astype(jnp.int32).reshape(B * L)
    tt16 = lax.bitcast_convert_type(
        text_table.astype(jnp.bfloat16), jnp.uint16)
    lo = tt16[:, :H2].astype(jnp.uint32)
    hi = tt16[:, H2:].astype(jnp.uint32)
    text_table_i32 = lax.bitcast_convert_type(lo | (hi << 16), jnp.int32)
    return _run(feat, ids, text_flat, text_len, W, b, id_table,
                text_table_i32)


# speedup vs baseline: 4.2360x; 1.0124x over previous
import dataclasses
import functools

import jax
import jax.numpy as jnp
from jax import lax
from jax.experimental import pallas as pl
from jax.experimental.pallas import tpu as pltpu
from jax.experimental.pallas import tpu_sc as plsc

B = 16384
FEAT_DIM = 256
H = 128
H2 = H // 2
L = 50
LH = L // 2

NC = 2
NS = 16
NW = NC * NS
B_PER_W = B // NW
CH = 16
N_CHUNKS = B_PER_W // CH
IDC = 128
N_IDC = B_PER_W // IDC

_MESH = plsc.VectorSubcoreMesh(
    core_axis_name="c", subcore_axis_name="s", num_cores=NC, num_subcores=NS
)


def _sc_text_body(text_hbm, text_table_hbm, recip_hbm, psum_hbm,
                  tidx_v, rows_v, acc_v, rcp_v, sems):
    (sem_g0, sem_g1, sem_x0, sem_x1, sem_o0, sem_o1) = sems
    sem_g = (sem_g0, sem_g1)
    sem_x = (sem_x0, sem_x1)
    sem_o = (sem_o0, sem_o1)

    wid = lax.axis_index("s") * NC + lax.axis_index("c")
    base = wid * B_PER_W

    for b in range(2):
        pltpu.sync_copy(text_hbm.at[pl.ds((base + b * CH) * L, CH * L)],
                        tidx_v[b])
        pltpu.sync_copy(recip_hbm.at[pl.ds(base + b * CH, CH)], rcp_v[b])
        pltpu.async_copy(text_table_hbm.at[tidx_v[b]], rows_v[b], sem_g[b])

    @pl.loop(0, N_CHUNKS, step=2)
    def _(c):
        for b in range(2):
            c2 = c + b
            pltpu.make_async_copy(
                text_table_hbm.at[tidx_v[b]], rows_v[b], sem_g[b]).wait()

            @pl.when(c2 + 2 < N_CHUNKS)
            def _():
                pltpu.async_copy(
                    text_hbm.at[pl.ds((base + (c2 + 2) * CH) * L, CH * L)],
                    tidx_v[b], sem_x[b])
                pltpu.async_copy(
                    recip_hbm.at[pl.ds(base + (c2 + 2) * CH, CH)],
                    rcp_v[b], sem_x[b])

            @pl.when(c2 >= 2)
            def _():
                pltpu.make_async_copy(
                    acc_v[b], psum_hbm.at[pl.ds(base, CH)], sem_o[b]).wait()

            recip16 = rcp_v[b][...]

            @pl.loop(0, CH)
            def _(e):
                def add_row(l, accs):
                    r = e * L + l
                    lo = tuple(
                        accs[h] + plsc.bitcast(
                            rows_v[b][r, pl.ds(h * 16, 16)], jnp.bfloat16)
                        for h in range(4)
                    )
                    hi = tuple(
                        accs[4 + h] + plsc.bitcast(
                            rows_v[b][r + LH, pl.ds(h * 16, 16)],
                            jnp.bfloat16)
                        for h in range(4)
                    )
                    return lo + hi

                accs = lax.fori_loop(
                    0, LH, add_row,
                    tuple(jnp.zeros((32,), jnp.bfloat16) for _ in range(8)),
                )
                mask = lax.iota(jnp.int32, 16) == e
                rs = jnp.sum(jnp.where(mask, recip16, 0.0))
                rsplat = plsc.pack(jnp.full((16,), rs, jnp.float32),
                                   jnp.full((16,), rs, jnp.float32),
                                   format=plsc.PackFormat.INTERLEAVED)
                for h in range(8):
                    acc_v[b][e, pl.ds(h * 16, 16)] = plsc.bitcast(
                        accs[h] * rsplat, jnp.int32)

            pltpu.async_copy(
                acc_v[b], psum_hbm.at[pl.ds(base + c2 * CH, CH)], sem_o[b])

            @pl.when(c2 + 2 < N_CHUNKS)
            def _():
                pltpu.make_async_copy(
                    text_hbm.at[pl.ds((base + (c2 + 2) * CH) * L, CH * L)],
                    tidx_v[b], sem_x[b]).wait()
                pltpu.make_async_copy(
                    recip_hbm.at[pl.ds(base + (c2 + 2) * CH, CH)],
                    rcp_v[b], sem_x[b]).wait()
                pltpu.async_copy(
                    text_table_hbm.at[tidx_v[b]], rows_v[b], sem_g[b])

    for b in range(2):
        pltpu.make_async_copy(
            acc_v[b], psum_hbm.at[pl.ds(base, CH)], sem_o[b]).wait()


def _sc_text(text_flat, text_table_i32, recip):
    out_type = jax.ShapeDtypeStruct((B, H), jnp.int32)
    scratch = [
        (pltpu.VMEM((CH * L,), jnp.int32),) * 2,
        (pltpu.VMEM((CH * L, H2), jnp.int32),) * 2,
        (pltpu.VMEM((CH, H), jnp.int32),) * 2,
        (pltpu.VMEM((CH,), jnp.float32),) * 2,
        (pltpu.SemaphoreType.DMA,) * 6,
    ]
    cp = pltpu.CompilerParams()
    if "needs_layout_passes" in pltpu.CompilerParams.__dataclass_fields__:
        cp = dataclasses.replace(cp, needs_layout_passes=False)
    if "use_tc_tiling_on_sc" in pltpu.CompilerParams.__dataclass_fields__:
        cp = dataclasses.replace(cp, use_tc_tiling_on_sc=False)
    return pl.kernel(
        _sc_text_body, out_type=out_type, mesh=_MESH, scratch_types=scratch,
        compiler_params=cp,
    )(text_flat, text_table_i32, recip)


def _sc_id_body(ids_hbm, id_table_hbm, pid_hbm, iidx_v, idrows_v, sem):
    wid = lax.axis_index("s") * NC + lax.axis_index("c")
    base = wid * B_PER_W

    @pl.loop(0, N_IDC)
    def _(k):
        row0 = base + k * IDC
        pltpu.sync_copy(ids_hbm.at[pl.ds(row0, IDC)], iidx_v)
        pltpu.async_copy(id_table_hbm.at[iidx_v], idrows_v, sem).wait()
        pltpu.sync_copy(idrows_v, pid_hbm.at[pl.ds(row0, IDC)])


def _sc_id(ids, id_table):
    out_type = jax.ShapeDtypeStruct((B, H), jnp.float32)
    scratch = [
        pltpu.VMEM((IDC,), jnp.int32),
        pltpu.VMEM((IDC, H), jnp.float32),
        pltpu.SemaphoreType.DMA,
    ]
    return pl.kernel(
        _sc_id_body, out_type=out_type, mesh=_MESH, scratch_types=scratch,
    )(ids, id_table)


BLK = 1024


def _mm_body(feat_ref, w_ref, b_ref, out_ref):
    out_ref[...] = lax.dot_general(
        feat_ref[...], w_ref[...], (((1,), (1,)), ((), ())),
        preferred_element_type=jnp.float32,
    ) + b_ref[...]


def _mm(feat, W, b2):
    return pl.pallas_call(
        _mm_body,
        grid=(B // BLK,),
        compiler_params=pltpu.CompilerParams(
            dimension_semantics=("parallel",)),
        in_specs=[
            pl.BlockSpec((BLK, FEAT_DIM), lambda i: (i, 0)),
            pl.BlockSpec((H, FEAT_DIM), lambda i: (0, 0)),
            pl.BlockSpec((1, H), lambda i: (0, 0)),
        ],
        out_specs=pl.BlockSpec((BLK, H), lambda i: (i, 0)),
        out_shape=jax.ShapeDtypeStruct((B, H), jnp.float32),
    )(feat, W, b2)


def _unpack_f32(w):
    lo = lax.bitcast_convert_type(w << 16, jnp.float32)
    hi = lax.bitcast_convert_type(
        w & jnp.int32(-65536), jnp.float32)
    return lo, hi


def _combine_body(mm_ref, pid_ref, ps_ref, out_ref):
    wa = ps_ref[:, :H2]
    wb = ps_ref[:, H2:]
    a_lo, a_hi = _unpack_f32(wa)
    b_lo, b_hi = _unpack_f32(wb)
    psum = jnp.concatenate([a_lo + b_lo, a_hi + b_hi], axis=1)
    out_ref[...] = mm_ref[...] + pid_ref[...] + psum


def _combine(mm, pid, psum_i32):
    return pl.pallas_call(
        _combine_body,
        grid=(B // BLK,),
        compiler_params=pltpu.CompilerParams(
            dimension_semantics=("parallel",)),
        in_specs=[
            pl.BlockSpec((BLK, H), lambda i: (i, 0)),
            pl.BlockSpec((BLK, H), lambda i: (i, 0)),
            pl.BlockSpec((BLK, H), lambda i: (i, 0)),
        ],
        out_specs=pl.BlockSpec((BLK, H), lambda i: (i, 0)),
        out_shape=jax.ShapeDtypeStruct((B, H), jnp.float32),
    )(mm, pid, psum_i32)


@jax.jit
def _run(feat, ids, text_flat, text_len, W, b, id_table, text_table_i32):
    pid = _sc_id(ids, id_table)
    recip = 1.0 / text_len.astype(jnp.float32)
    psum_i32 = _sc_text(text_flat, text_table_i32, recip)
    b2 = b.reshape(1, H)
    mm = _mm(feat, W, b2)
    return _combine(mm, pid, psum_i32)


def kernel(feat, ids, text, text_len, W, b, id_table, text_table):
    ids = ids.astype(jnp.int32)
    text_flat = text.astype(jnp.int32).reshape(B * L)
    tt16 = lax.bitcast_convert_type(
        text_table.astype(jnp.bfloat16), jnp.uint16)
    lo = tt16[:, :H2].astype(jnp.uint32)
    hi = tt16[:, H2:].astype(jnp.uint32)
    text_table_i32 = lax.bitcast_convert_type(lo | (hi << 16), jnp.int32)
    return _run(feat, ids, text_flat, text_len, W, b, id_table,
                text_table_i32)


# speedup vs baseline: 4.4974x; 1.0617x over previous
import dataclasses
import functools

import jax
import jax.numpy as jnp
from jax import lax
from jax.experimental import pallas as pl
from jax.experimental.pallas import tpu as pltpu
from jax.experimental.pallas import tpu_sc as plsc

B = 16384
FEAT_DIM = 256
H = 128
H2 = H // 2
L = 50
LH = L // 2

NC = 2
NS = 16
NW = NC * NS
B_PER_W = B // NW
CH = 16
N_CHUNKS = B_PER_W // CH
IDC = 128
N_IDC = B_PER_W // IDC

_MESH = plsc.VectorSubcoreMesh(
    core_axis_name="c", subcore_axis_name="s", num_cores=NC, num_subcores=NS
)


def _sc_text_body(text_hbm, text_table_hbm, recip_hbm, psum_hbm,
                  tidx_v, rows_v, acc_v, rcp_v, sems):
    (sem_g0, sem_g1, sem_x0, sem_x1, sem_o0, sem_o1) = sems
    sem_g = (sem_g0, sem_g1)
    sem_x = (sem_x0, sem_x1)
    sem_o = (sem_o0, sem_o1)

    wid = lax.axis_index("s") * NC + lax.axis_index("c")
    base = wid * B_PER_W

    for b in range(2):
        pltpu.sync_copy(text_hbm.at[pl.ds((base + b * CH) * L, CH * L)],
                        tidx_v[b])
        pltpu.sync_copy(recip_hbm.at[pl.ds(base + b * CH, CH)], rcp_v[b])
        pltpu.async_copy(text_table_hbm.at[tidx_v[b]], rows_v[b], sem_g[b])

    @pl.loop(0, N_CHUNKS, step=2)
    def _(c):
        for b in range(2):
            c2 = c + b
            pltpu.make_async_copy(
                text_table_hbm.at[tidx_v[b]], rows_v[b], sem_g[b]).wait()

            @pl.when(c2 + 2 < N_CHUNKS)
            def _():
                pltpu.async_copy(
                    text_hbm.at[pl.ds((base + (c2 + 2) * CH) * L, CH * L)],
                    tidx_v[b], sem_x[b])
                pltpu.async_copy(
                    recip_hbm.at[pl.ds(base + (c2 + 2) * CH, CH)],
                    rcp_v[b], sem_x[b])

            @pl.when(c2 >= 2)
            def _():
                pltpu.make_async_copy(
                    acc_v[b], psum_hbm.at[pl.ds(base, CH)], sem_o[b]).wait()

            recip16 = rcp_v[b][...]

            @pl.loop(0, CH)
            def _(e):
                def add_row(l, accs):
                    r = e * L + l
                    lo = tuple(
                        accs[h] + plsc.bitcast(
                            rows_v[b][r, pl.ds(h * 16, 16)], jnp.bfloat16)
                        for h in range(4)
                    )
                    hi = tuple(
                        accs[4 + h] + plsc.bitcast(
                            rows_v[b][r + LH, pl.ds(h * 16, 16)],
                            jnp.bfloat16)
                        for h in range(4)
                    )
                    return lo + hi

                accs = lax.fori_loop(
                    0, LH, add_row,
                    tuple(jnp.zeros((32,), jnp.bfloat16) for _ in range(8)),
                )
                mask = lax.iota(jnp.int32, 16) == e
                rs = jnp.sum(jnp.where(mask, recip16, 0.0))
                rsplat = plsc.pack(jnp.full((16,), rs, jnp.float32),
                                   jnp.full((16,), rs, jnp.float32),
                                   format=plsc.PackFormat.INTERLEAVED)
                for h in range(8):
                    acc_v[b][e, pl.ds(h * 16, 16)] = plsc.bitcast(
                        accs[h] * rsplat, jnp.int32)

            pltpu.async_copy(
                acc_v[b], psum_hbm.at[pl.ds(base + c2 * CH, CH)], sem_o[b])

            @pl.when(c2 + 2 < N_CHUNKS)
            def _():
                pltpu.make_async_copy(
                    text_hbm.at[pl.ds((base + (c2 + 2) * CH) * L, CH * L)],
                    tidx_v[b], sem_x[b]).wait()
                pltpu.make_async_copy(
                    recip_hbm.at[pl.ds(base + (c2 + 2) * CH, CH)],
                    rcp_v[b], sem_x[b]).wait()
                pltpu.async_copy(
                    text_table_hbm.at[tidx_v[b]], rows_v[b], sem_g[b])

    for b in range(2):
        pltpu.make_async_copy(
            acc_v[b], psum_hbm.at[pl.ds(base, CH)], sem_o[b]).wait()


def _sc_text(text_flat, text_table_i32, recip):
    out_type = jax.ShapeDtypeStruct((B, H), jnp.int32)
    scratch = [
        (pltpu.VMEM((CH * L,), jnp.int32),) * 2,
        (pltpu.VMEM((CH * L, H2), jnp.int32),) * 2,
        (pltpu.VMEM((CH, H), jnp.int32),) * 2,
        (pltpu.VMEM((CH,), jnp.float32),) * 2,
        (pltpu.SemaphoreType.DMA,) * 6,
    ]
    cp = pltpu.CompilerParams()
    if "needs_layout_passes" in pltpu.CompilerParams.__dataclass_fields__:
        cp = dataclasses.replace(cp, needs_layout_passes=False)
    if "use_tc_tiling_on_sc" in pltpu.CompilerParams.__dataclass_fields__:
        cp = dataclasses.replace(cp, use_tc_tiling_on_sc=False)
    return pl.kernel(
        _sc_text_body, out_type=out_type, mesh=_MESH, scratch_types=scratch,
        compiler_params=cp,
    )(text_flat, text_table_i32, recip)


def _sc_id_body(ids_hbm, id_table_hbm, pid_hbm, iidx_v, idrows_v, sem):
    wid = lax.axis_index("s") * NC + lax.axis_index("c")
    base = wid * B_PER_W

    @pl.loop(0, N_IDC)
    def _(k):
        row0 = base + k * IDC
        pltpu.sync_copy(ids_hbm.at[pl.ds(row0, IDC)], iidx_v)
        pltpu.async_copy(id_table_hbm.at[iidx_v], idrows_v, sem).wait()
        pltpu.sync_copy(idrows_v, pid_hbm.at[pl.ds(row0, IDC)])


def _sc_id(ids, id_table):
    out_type = jax.ShapeDtypeStruct((B, H), jnp.float32)
    scratch = [
        pltpu.VMEM((IDC,), jnp.int32),
        pltpu.VMEM((IDC, H), jnp.float32),
        pltpu.SemaphoreType.DMA,
    ]
    return pl.kernel(
        _sc_id_body, out_type=out_type, mesh=_MESH, scratch_types=scratch,
    )(ids, id_table)


BLK = 1024


def _mm_body(feat_ref, w_ref, b_ref, out_ref):
    out_ref[...] = lax.dot_general(
        feat_ref[...], w_ref[...], (((1,), (1,)), ((), ())),
        preferred_element_type=jnp.float32,
    ) + b_ref[...]


def _mm(feat, W, b2):
    return pl.pallas_call(
        _mm_body,
        grid=(B // BLK,),
        compiler_params=pltpu.CompilerParams(
            dimension_semantics=("parallel",)),
        in_specs=[
            pl.BlockSpec((BLK, FEAT_DIM), lambda i: (i, 0)),
            pl.BlockSpec((H, FEAT_DIM), lambda i: (0, 0)),
            pl.BlockSpec((1, H), lambda i: (0, 0)),
        ],
        out_specs=pl.BlockSpec((BLK, H), lambda i: (i, 0)),
        out_shape=jax.ShapeDtypeStruct((B, H), jnp.float32),
    )(feat, W, b2)


def _unpack_f32(w):
    lo = lax.bitcast_convert_type(w << 16, jnp.float32)
    hi = lax.bitcast_convert_type(
        w & jnp.int32(-65536), jnp.float32)
    return lo, hi


def _combine_body(mm_ref, pid_ref, ps_ref, out_ref):
    wa = ps_ref[:, :H2]
    wb = ps_ref[:, H2:]
    a_lo, a_hi = _unpack_f32(wa)
    b_lo, b_hi = _unpack_f32(wb)
    psum = jnp.concatenate([a_lo + b_lo, a_hi + b_hi], axis=1)
    out_ref[...] = mm_ref[...] + pid_ref[...] + psum


def _combine(mm, pid, psum_i32):
    return pl.pallas_call(
        _combine_body,
        grid=(B // BLK,),
        compiler_params=pltpu.CompilerParams(
            dimension_semantics=("parallel",)),
        in_specs=[
            pl.BlockSpec((BLK, H), lambda i: (i, 0)),
            pl.BlockSpec((BLK, H), lambda i: (i, 0)),
            pl.BlockSpec((BLK, H), lambda i: (i, 0)),
        ],
        out_specs=pl.BlockSpec((BLK, H), lambda i: (i, 0)),
        out_shape=jax.ShapeDtypeStruct((B, H), jnp.float32),
    )(mm, pid, psum_i32)


@jax.jit
def _run(feat, ids, text_flat, text_len, W, b, id_table, text_table_i32):
    pid = _sc_id(ids, id_table)
    recip = 1.0 / text_len.astype(jnp.float32)
    text_flat, text_table_i32, recip, pid = lax.optimization_barrier(
        (text_flat, text_table_i32, recip, pid))
    psum_i32 = _sc_text(text_flat, text_table_i32, recip)
    b2 = b.reshape(1, H)
    mm = _mm(feat, W, b2)
    return _combine(mm, pid, psum_i32)


def kernel(feat, ids, text, text_len, W, b, id_table, text_table):
    ids = ids.astype(jnp.int32)
    text_flat = text.astype(jnp.int32).reshape(B * L)
    tt16 = lax.bitcast_convert_type(
        text_table.astype(jnp.bfloat16), jnp.uint16)
    lo = tt16[:, :H2].astype(jnp.uint32)
    hi = tt16[:, H2:].astype(jnp.uint32)
    text_table_i32 = lax.bitcast_convert_type(lo | (hi << 16), jnp.int32)
    return _run(feat, ids, text_flat, text_len, W, b, id_table,
                text_table_i32)
